# baseline (speedup 1.0000x reference)
import functools

import jax
import jax.numpy as jnp
from jax import lax
from jax.experimental import pallas as pl
from jax.experimental.pallas import tpu as pltpu
from jax.experimental.pallas import tpu_sc as plsc

EPS_ = 1e-4
B_, J_, D_, L_ = 8, 2048, 512, 4096
CHUNK = 256
NCH_ = J_ // CHUNK
BCHUNK = 512
NBC_ = L_ // BCHUNK
NW = 32
G_ = 2
BG_ = B_ // G_
RPW_ = (BG_ * L_) // NW
GCHUNK = 64
NGCG_ = RPW_ // GCHUNK


def _ema_idx_kernel(emb_ref, conf_ref, mask_ref, bnd_ref, sm_ref, idx_ref):
    f32 = jnp.float32
    dn_m = (((1,), (0,)), ((), ()))
    dn_t = (((1,), (1,)), ((), ()))
    HI = lax.Precision.HIGHEST
    DF = lax.Precision.DEFAULT

    C = CHUNK
    ri = lax.broadcasted_iota(jnp.int32, (C, C), 0)
    ci = lax.broadcasted_iota(jnp.int32, (C, C), 1)
    triu = (ri <= ci).astype(f32)
    eye = (ri == ci).astype(f32)

    conf = conf_ref[0]
    msk = mask_ref[0]
    p = jnp.clip(conf, EPS_, 1.0 - EPS_)
    valid = msk > 0.5
    la = jnp.log(jnp.where(valid, 1.0 - p, 1.0))
    bcoef = jnp.where(valid, p, 0.0)
    cum = lax.dot_general(la, triu, dn_m, precision=HI,
                          preferred_element_type=f32)
    cumt = lax.dot_general(eye, cum, dn_t, precision=HI,
                           preferred_element_type=f32)
    bt = lax.dot_general(eye, bcoef, dn_t, precision=HI,
                         preferred_element_type=f32)
    ecolt = jnp.exp(cumt)

    carry = jnp.zeros((1, D_), f32)
    for c in range(NCH_):
        m = jnp.where(ri >= ci, jnp.exp(cumt[:, c:c + 1] - cum[c:c + 1, :]),
                      0.0)
        bmat = bt[:, c:c + 1] * emb_ref[0, c * C:(c + 1) * C, :]
        sm = lax.dot_general(m, bmat, dn_m, precision=DF,
                             preferred_element_type=f32)
        sm = sm + ecolt[:, c:c + 1] * carry
        sm_ref[0, c * C:(c + 1) * C, :] = sm
        carry = sm[C - 1:C, :]

    CB = BCHUNK
    rb = lax.broadcasted_iota(jnp.int32, (CB, CB), 0)
    cbi = lax.broadcasted_iota(jnp.int32, (CB, CB), 1)
    triu_b = (rb <= cbi).astype(f32)
    r8 = lax.broadcasted_iota(jnp.int32, (NBC_, NBC_), 0)
    c8 = lax.broadcasted_iota(jnp.int32, (NBC_, NBC_), 1)
    trilx = (r8 > c8).astype(f32)

    bnd = bnd_ref[0]
    cumb = lax.dot_general(bnd, triu_b, dn_m, precision=DF,
                           preferred_element_type=f32)
    tot = cumb[:, CB - 1:CB]
    pre = lax.dot_general(trilx, tot, dn_m, precision=lax.Precision.HIGHEST,
                          preferred_element_type=f32)
    cum_all = cumb + pre
    idx = jnp.clip(cum_all - 1.0, 0.0, float(J_ - 1)).astype(jnp.int32)
    idx_ref[0] = idx + pl.program_id(0) * J_


def _ema_idx(emb, conf, msk, bnd):
    nb = emb.shape[0]
    return pl.pallas_call(
        _ema_idx_kernel,
        grid=(nb,),
        in_specs=[
            pl.BlockSpec((1, J_, D_), lambda b: (b, 0, 0)),
            pl.BlockSpec((1, NCH_, CHUNK), lambda b: (b, 0, 0)),
            pl.BlockSpec((1, NCH_, CHUNK), lambda b: (b, 0, 0)),
            pl.BlockSpec((1, NBC_, BCHUNK), lambda b: (b, 0, 0)),
        ],
        out_specs=[
            pl.BlockSpec((1, J_, D_), lambda b: (b, 0, 0)),
            pl.BlockSpec((1, NBC_, BCHUNK), lambda b: (b, 0, 0)),
        ],
        out_shape=[
            jax.ShapeDtypeStruct((nb, J_, D_), jnp.float32),
            jax.ShapeDtypeStruct((nb, NBC_, BCHUNK), jnp.int32),
        ],
    )(emb, conf, msk, bnd)


def _make_sc_gather(out_base, first):
    mesh = plsc.VectorSubcoreMesh(core_axis_name="c", subcore_axis_name="s")
    out_type = jax.ShapeDtypeStruct((B_ * L_, D_), jnp.float32) if first else ()

    @functools.partial(
        pl.kernel,
        mesh=mesh,
        out_type=out_type,
        scratch_types=[
            pltpu.VMEM((NGCG_, GCHUNK), jnp.int32),
            pltpu.VMEM((2, GCHUNK, D_), jnp.float32),
            pltpu.SemaphoreType.DMA,
            pltpu.SemaphoreType.DMA,
        ],
    )
    def k(table_hbm, idx_hbm, out_hbm, idx_v, rows_v, semg, semw):
        wid = lax.axis_index("s") * 2 + lax.axis_index("c")
        base = out_base + wid * RPW_

        def wslice(c):
            return out_hbm.at[pl.ds(base + c * GCHUNK, GCHUNK)]

        pltpu.sync_copy(idx_hbm.at[wid], idx_v)
        pltpu.async_copy(table_hbm.at[idx_v.at[0]], rows_v.at[0], semg)
        for c in range(NGCG_):
            p = c % 2
            pltpu.make_async_copy(table_hbm.at[idx_v.at[c]],
                                  rows_v.at[p], semg).wait()
            if c + 1 < NGCG_:
                if c >= 1:
                    pltpu.make_async_copy(rows_v.at[1 - p],
                                          wslice(c - 1), semw).wait()
                pltpu.async_copy(table_hbm.at[idx_v.at[c + 1]],
                                 rows_v.at[1 - p], semg)
            pltpu.async_copy(rows_v.at[p], wslice(c), semw)
        pltpu.make_async_copy(rows_v.at[0], wslice(NGCG_ - 2), semw).wait()
        pltpu.make_async_copy(rows_v.at[1], wslice(NGCG_ - 1), semw).wait()

    return k


def kernel(unit_embeddings, unit_confidence, unit_mask, boundary_mask):
    conf = unit_confidence.reshape(B_, NCH_, CHUNK)
    msk = unit_mask.astype(jnp.float32).reshape(B_, NCH_, CHUNK)
    bnd = boundary_mask.astype(jnp.float32).reshape(B_, NBC_, BCHUNK)
    sms, idxs = [], []
    for g in range(G_):
        s = slice(g * BG_, (g + 1) * BG_)
        sm_g, idx_g = _ema_idx(unit_embeddings[s], conf[s], msk[s], bnd[s])
        sms.append(sm_g.reshape(BG_ * J_, D_))
        idxs.append(idx_g.reshape(NW, NGCG_, GCHUNK))
    out0 = _make_sc_gather(0, True)(sms[0], idxs[0])
    ref = jax.new_ref(out0)
    for g in range(1, G_):
        _make_sc_gather(g * BG_ * L_, False)(sms[g], idxs[g], ref)
    return ref[...].reshape(B_, L_, D_)


# speedup vs baseline: 1.2907x; 1.2907x over previous
import functools

import jax
import jax.numpy as jnp
from jax import lax
from jax.experimental import pallas as pl
from jax.experimental.pallas import tpu as pltpu
from jax.experimental.pallas import tpu_sc as plsc

EPS_ = 1e-4
B_, J_, D_, L_ = 8, 2048, 512, 4096
CHUNK = 128
NCH_ = J_ // CHUNK
BCHUNK = 512
NBC_ = L_ // BCHUNK
NW = 32
RPW_ = (B_ * L_) // NW
GCHUNK = 128
NGCG_ = RPW_ // GCHUNK


def _ema_idx_kernel(emb_ref, conf_ref, mask_ref, bnd_ref, sm_ref, idx_ref):
    f32 = jnp.float32
    dn_m = (((1,), (0,)), ((), ()))
    dn_t = (((1,), (1,)), ((), ()))
    HI = lax.Precision.HIGHEST
    DF = lax.Precision.DEFAULT

    C = CHUNK
    ri = lax.broadcasted_iota(jnp.int32, (C, C), 0)
    ci = lax.broadcasted_iota(jnp.int32, (C, C), 1)
    triu = (ri <= ci).astype(f32)
    eye = (ri == ci).astype(f32)

    conf = conf_ref[0]
    msk = mask_ref[0]
    p = jnp.clip(conf, EPS_, 1.0 - EPS_)
    valid = msk > 0.5
    la = jnp.log(jnp.where(valid, 1.0 - p, 1.0))
    bcoef = jnp.where(valid, p, 0.0)
    cum = lax.dot_general(la, triu, dn_m, precision=HI,
                          preferred_element_type=f32)
    cumt = lax.dot_general(eye, cum, dn_t, precision=HI,
                           preferred_element_type=f32)
    bt = lax.dot_general(eye, bcoef, dn_t, precision=HI,
                         preferred_element_type=f32)
    ecolt = jnp.exp(cumt)

    carry = jnp.zeros((1, D_), f32)
    for c in range(NCH_):
        m = jnp.where(ri >= ci, jnp.exp(cumt[:, c:c + 1] - cum[c:c + 1, :]),
                      0.0)
        bmat = bt[:, c:c + 1] * emb_ref[0, c * C:(c + 1) * C, :]
        sm = lax.dot_general(m, bmat, dn_m, precision=DF,
                             preferred_element_type=f32)
        sm = sm + ecolt[:, c:c + 1] * carry
        sm_ref[0, c * C:(c + 1) * C, :] = sm
        carry = sm[C - 1:C, :]

    CB = BCHUNK
    rb = lax.broadcasted_iota(jnp.int32, (CB, CB), 0)
    cbi = lax.broadcasted_iota(jnp.int32, (CB, CB), 1)
    triu_b = (rb <= cbi).astype(f32)
    r8 = lax.broadcasted_iota(jnp.int32, (NBC_, NBC_), 0)
    c8 = lax.broadcasted_iota(jnp.int32, (NBC_, NBC_), 1)
    trilx = (r8 > c8).astype(f32)

    bnd = bnd_ref[0]
    cumb = lax.dot_general(bnd, triu_b, dn_m, precision=DF,
                           preferred_element_type=f32)
    tot = cumb[:, CB - 1:CB]
    pre = lax.dot_general(trilx, tot, dn_m, precision=lax.Precision.HIGHEST,
                          preferred_element_type=f32)
    cum_all = cumb + pre
    idx = jnp.clip(cum_all - 1.0, 0.0, float(J_ - 1)).astype(jnp.int32)
    idx_ref[0] = idx + pl.program_id(0) * J_


def _ema_idx(emb, conf, msk, bnd):
    nb = emb.shape[0]
    return pl.pallas_call(
        _ema_idx_kernel,
        grid=(nb,),
        in_specs=[
            pl.BlockSpec((1, J_, D_), lambda b: (b, 0, 0)),
            pl.BlockSpec((1, NCH_, CHUNK), lambda b: (b, 0, 0)),
            pl.BlockSpec((1, NCH_, CHUNK), lambda b: (b, 0, 0)),
            pl.BlockSpec((1, NBC_, BCHUNK), lambda b: (b, 0, 0)),
        ],
        out_specs=[
            pl.BlockSpec((1, J_, D_), lambda b: (b, 0, 0)),
            pl.BlockSpec((1, NBC_, BCHUNK), lambda b: (b, 0, 0)),
        ],
        out_shape=[
            jax.ShapeDtypeStruct((nb, J_, D_), jnp.float32),
            jax.ShapeDtypeStruct((nb, NBC_, BCHUNK), jnp.int32),
        ],
    )(emb, conf, msk, bnd)


def _sc_gather(table, idx3):
    mesh = plsc.VectorSubcoreMesh(core_axis_name="c", subcore_axis_name="s")

    @functools.partial(
        pl.kernel,
        mesh=mesh,
        out_type=jax.ShapeDtypeStruct((B_ * L_, D_), jnp.float32),
        scratch_types=[
            pltpu.VMEM((NGCG_, GCHUNK), jnp.int32),
            pltpu.VMEM((GCHUNK, D_), jnp.float32),
            pltpu.SemaphoreType.DMA,
        ],
    )
    def k(table_hbm, idx_hbm, out_hbm, idx_v, rows_v, sem):
        wid = lax.axis_index("s") * 2 + lax.axis_index("c")
        base = wid * RPW_
        pltpu.sync_copy(idx_hbm.at[wid], idx_v)

        def body(c, carry):
            pltpu.async_copy(table_hbm.at[idx_v.at[c]], rows_v, sem).wait()
            pltpu.sync_copy(rows_v,
                            out_hbm.at[pl.ds(base + c * GCHUNK, GCHUNK)])
            return carry

        lax.fori_loop(0, NGCG_, body, 0)

    return k(table, idx3)


def kernel(unit_embeddings, unit_confidence, unit_mask, boundary_mask):
    conf = unit_confidence.reshape(B_, NCH_, CHUNK)
    msk = unit_mask.astype(jnp.float32).reshape(B_, NCH_, CHUNK)
    bnd = boundary_mask.astype(jnp.float32).reshape(B_, NBC_, BCHUNK)
    smoothed, idx = _ema_idx(unit_embeddings, conf, msk, bnd)
    frames = _sc_gather(smoothed.reshape(B_ * J_, D_),
                        idx.reshape(NW, NGCG_, GCHUNK))
    return frames.reshape(B_, L_, D_)
